# baseline (speedup 1.0000x reference)
import functools

import jax
import jax.numpy as jnp
from jax import lax
from jax.experimental import pallas as pl
from jax.experimental.pallas import tpu as pltpu
from jax.experimental.pallas import tpu_sc as plsc

_N_FIELDS = 26
_EMBED_DIM = 64
_N_PAIRS = (_N_FIELDS * (_N_FIELDS - 1)) // 2
_LANES = 16


def _first_order_sc(sf_t, w_flat):
    B = sf_t.shape[1]
    info = plsc.get_sparse_core_info()
    nw = info.num_cores * info.num_subcores
    bpw = B // nw
    mesh = plsc.VectorSubcoreMesh(core_axis_name="c", subcore_axis_name="s")

    @functools.partial(
        pl.kernel,
        mesh=mesh,
        out_type=jax.ShapeDtypeStruct((B,), jnp.float32),
        scratch_types=[
            pltpu.VMEM((_N_FIELDS, bpw), jnp.int32),
            pltpu.VMEM((_N_FIELDS, bpw), jnp.float32),
            pltpu.VMEM((bpw,), jnp.float32),
            pltpu.SemaphoreType.DMA,
        ],
    )
    def fo(sf_hbm, w_hbm, out_hbm, idx_v, vals_v, acc_v, sem):
        wid = lax.axis_index("s") * info.num_cores + lax.axis_index("c")
        base = wid * bpw
        pltpu.sync_copy(sf_hbm.at[:, pl.ds(base, bpw)], idx_v)
        cops = [
            pltpu.async_copy(w_hbm.at[idx_v.at[f]], vals_v.at[f], sem)
            for f in range(_N_FIELDS)
        ]
        for c in cops:
            c.wait()
        for c in range(bpw // _LANES):
            s = pl.ds(c * _LANES, _LANES)
            acc = vals_v[0, s]
            for f in range(1, _N_FIELDS):
                acc = acc + vals_v[f, s]
            acc_v[s] = acc
        pltpu.sync_copy(acc_v, out_hbm.at[pl.ds(base, bpw)])

    return fo(sf_t, w_flat)


_PAIRS = [
    (i, j) for i in range(_N_FIELDS) for j in range(i + 1, _N_FIELDS)
]


def _second_order_body2d(in_ref, out_ref):
    d = _EMBED_DIM
    fields = [in_ref[:, i * d : (i + 1) * d] for i in range(_N_FIELDS)]
    for p, (i, j) in enumerate(_PAIRS):
        out_ref[:, p * d : (p + 1) * d] = fields[i] * fields[j]


def _second_order_tc2d(embed_stack, tb):
    B = embed_stack.shape[0]
    in2 = embed_stack.reshape(B, _N_FIELDS * _EMBED_DIM)
    out2 = pl.pallas_call(
        _second_order_body2d,
        grid=(B // tb,),
        in_specs=[pl.BlockSpec((tb, _N_FIELDS * _EMBED_DIM), lambda b: (b, 0))],
        out_specs=pl.BlockSpec((tb, _N_PAIRS * _EMBED_DIM), lambda b: (b, 0)),
        out_shape=jax.ShapeDtypeStruct((B, _N_PAIRS * _EMBED_DIM), jnp.float32),
    )(in2)
    return out2.reshape(B, _N_PAIRS, _EMBED_DIM)


def _second_order_sc(embed2d, base_row, n_rows):
    d2 = _N_FIELDS * _EMBED_DIM
    dp = _N_PAIRS * _EMBED_DIM
    info = plsc.get_sparse_core_info()
    nw = info.num_cores * info.num_subcores
    bpw = n_rows // nw
    mesh = plsc.VectorSubcoreMesh(core_axis_name="c", subcore_axis_name="s")

    @functools.partial(
        pl.kernel,
        mesh=mesh,
        out_type=jax.ShapeDtypeStruct((n_rows * dp,), jnp.float32),
        scratch_types=[
            pltpu.VMEM((1, d2), jnp.float32),
            pltpu.VMEM((dp,), jnp.float32),
        ],
    )
    def so(emb_hbm, out_hbm, in_v, out_v):
        wid = lax.axis_index("s") * info.num_cores + lax.axis_index("c")
        row0 = wid * bpw

        def body(k, carry):
            pltpu.sync_copy(
                emb_hbm.at[pl.ds(base_row + row0 + k, 1), :], in_v
            )
            for c in range(_EMBED_DIM // _LANES):
                off = c * _LANES
                fields = [
                    in_v[0, pl.ds(f * _EMBED_DIM + off, _LANES)]
                    for f in range(_N_FIELDS)
                ]
                for p, (i, j) in enumerate(_PAIRS):
                    out_v[pl.ds(p * _EMBED_DIM + off, _LANES)] = (
                        fields[i] * fields[j]
                    )
            pltpu.sync_copy(out_v, out_hbm.at[pl.ds((row0 + k) * dp, dp)])
            return carry

        lax.fori_loop(0, bpw, body, None)

    return so(embed2d)


def _second_order_tc_partial(in2, n_rows, tb):
    out2 = pl.pallas_call(
        _second_order_body2d,
        grid=(n_rows // tb,),
        in_specs=[pl.BlockSpec((tb, _N_FIELDS * _EMBED_DIM), lambda b: (b, 0))],
        out_specs=pl.BlockSpec((tb, _N_PAIRS * _EMBED_DIM), lambda b: (b, 0)),
        out_shape=jax.ShapeDtypeStruct((n_rows, _N_PAIRS * _EMBED_DIM), jnp.float32),
    )(in2)
    return out2


@jax.jit
def kernel(embed_stack, sparse_feat, w):
    B = embed_stack.shape[0]
    sf_t = sparse_feat.T
    w_flat = w.reshape(-1)
    first = _first_order_sc(sf_t, w_flat).reshape(B, 1)
    in2 = embed_stack.reshape(B, _N_FIELDS * _EMBED_DIM)
    second = _second_order_sc(in2, 0, B).reshape(B, _N_PAIRS, _EMBED_DIM)
    return (first, second)


# speedup vs baseline: 2.3085x; 2.3085x over previous
import functools

import jax
import jax.numpy as jnp
from jax import lax
from jax.experimental import pallas as pl
from jax.experimental.pallas import tpu as pltpu
from jax.experimental.pallas import tpu_sc as plsc

_N_FIELDS = 26
_EMBED_DIM = 64
_N_PAIRS = (_N_FIELDS * (_N_FIELDS - 1)) // 2
_LANES = 16


def _first_order_sc(sf_t, w_flat):
    B = sf_t.shape[1]
    info = plsc.get_sparse_core_info()
    nw = info.num_cores * info.num_subcores
    bpw = B // nw
    mesh = plsc.VectorSubcoreMesh(core_axis_name="c", subcore_axis_name="s")

    @functools.partial(
        pl.kernel,
        mesh=mesh,
        out_type=jax.ShapeDtypeStruct((B,), jnp.float32),
        scratch_types=[
            pltpu.VMEM((_N_FIELDS, bpw), jnp.int32),
            pltpu.VMEM((_N_FIELDS, bpw), jnp.float32),
            pltpu.VMEM((bpw,), jnp.float32),
            pltpu.SemaphoreType.DMA,
        ],
    )
    def fo(sf_hbm, w_hbm, out_hbm, idx_v, vals_v, acc_v, sem):
        wid = lax.axis_index("s") * info.num_cores + lax.axis_index("c")
        base = wid * bpw
        pltpu.sync_copy(sf_hbm.at[:, pl.ds(base, bpw)], idx_v)
        cops = [
            pltpu.async_copy(w_hbm.at[idx_v.at[f]], vals_v.at[f], sem)
            for f in range(_N_FIELDS)
        ]
        for c in cops:
            c.wait()
        for c in range(bpw // _LANES):
            s = pl.ds(c * _LANES, _LANES)
            acc = vals_v[0, s]
            for f in range(1, _N_FIELDS):
                acc = acc + vals_v[f, s]
            acc_v[s] = acc
        pltpu.sync_copy(acc_v, out_hbm.at[pl.ds(base, bpw)])

    return fo(sf_t, w_flat)


_PAIRS = [
    (i, j) for i in range(_N_FIELDS) for j in range(i + 1, _N_FIELDS)
]


def _second_order_body2d(in_ref, out_ref):
    d = _EMBED_DIM
    fields = [in_ref[:, i * d : (i + 1) * d] for i in range(_N_FIELDS)]
    for p, (i, j) in enumerate(_PAIRS):
        out_ref[:, p * d : (p + 1) * d] = fields[i] * fields[j]


def _second_order_tc2d(embed_stack, tb):
    B = embed_stack.shape[0]
    in2 = embed_stack.reshape(B, _N_FIELDS * _EMBED_DIM)
    out2 = pl.pallas_call(
        _second_order_body2d,
        grid=(B // tb,),
        in_specs=[pl.BlockSpec((tb, _N_FIELDS * _EMBED_DIM), lambda b: (b, 0))],
        out_specs=pl.BlockSpec((tb, _N_PAIRS * _EMBED_DIM), lambda b: (b, 0)),
        out_shape=jax.ShapeDtypeStruct((B, _N_PAIRS * _EMBED_DIM), jnp.float32),
        compiler_params=pltpu.CompilerParams(
            dimension_semantics=("parallel",)
        ),
    )(in2)
    return out2.reshape(B, _N_PAIRS, _EMBED_DIM)


def _second_order_sc(embed2d, base_row, n_rows):
    d2 = _N_FIELDS * _EMBED_DIM
    dp = _N_PAIRS * _EMBED_DIM
    info = plsc.get_sparse_core_info()
    nw = info.num_cores * info.num_subcores
    bpw = n_rows // nw
    mesh = plsc.VectorSubcoreMesh(core_axis_name="c", subcore_axis_name="s")

    @functools.partial(
        pl.kernel,
        mesh=mesh,
        out_type=jax.ShapeDtypeStruct((n_rows, dp), jnp.float32),
        scratch_types=[
            pltpu.VMEM((1, d2), jnp.float32),
            pltpu.VMEM((1, dp), jnp.float32),
        ],
    )
    def so(emb_hbm, out_hbm, in_v, out_v):
        wid = lax.axis_index("s") * info.num_cores + lax.axis_index("c")
        row0 = wid * bpw

        def body(k, carry):
            pltpu.sync_copy(
                emb_hbm.at[pl.ds(base_row + row0 + k, 1), :], in_v
            )
            for c in range(_EMBED_DIM // _LANES):
                off = c * _LANES
                fields = [
                    in_v[0, pl.ds(f * _EMBED_DIM + off, _LANES)]
                    for f in range(_N_FIELDS)
                ]
                for p, (i, j) in enumerate(_PAIRS):
                    out_v[0, pl.ds(p * _EMBED_DIM + off, _LANES)] = (
                        fields[i] * fields[j]
                    )
            pltpu.sync_copy(out_v, out_hbm.at[pl.ds(row0 + k, 1), :])
            return carry

        lax.fori_loop(0, bpw, body, None)

    return so(embed2d)


def _second_order_tc_partial(in2, n_rows, tb):
    out2 = pl.pallas_call(
        _second_order_body2d,
        grid=(n_rows // tb,),
        in_specs=[pl.BlockSpec((tb, _N_FIELDS * _EMBED_DIM), lambda b: (b, 0))],
        out_specs=pl.BlockSpec((tb, _N_PAIRS * _EMBED_DIM), lambda b: (b, 0)),
        out_shape=jax.ShapeDtypeStruct((n_rows, _N_PAIRS * _EMBED_DIM), jnp.float32),
    )(in2)
    return out2


@jax.jit
def kernel(embed_stack, sparse_feat, w):
    B = embed_stack.shape[0]
    sf_t = sparse_feat.T
    w_flat = w.reshape(-1)
    first = _first_order_sc(sf_t, w_flat).reshape(B, 1)
    second = _second_order_tc2d(embed_stack, 128)
    return (first, second)


# speedup vs baseline: 2.3106x; 1.0009x over previous
import functools

import jax
import jax.numpy as jnp
from jax import lax
from jax.experimental import pallas as pl
from jax.experimental.pallas import tpu as pltpu
from jax.experimental.pallas import tpu_sc as plsc

_N_FIELDS = 26
_EMBED_DIM = 64
_N_PAIRS = (_N_FIELDS * (_N_FIELDS - 1)) // 2
_LANES = 16


def _first_order_sc(sf_t, w_flat):
    B = sf_t.shape[1]
    info = plsc.get_sparse_core_info()
    nw = info.num_cores * info.num_subcores
    bpw = B // nw
    mesh = plsc.VectorSubcoreMesh(core_axis_name="c", subcore_axis_name="s")

    @functools.partial(
        pl.kernel,
        mesh=mesh,
        out_type=jax.ShapeDtypeStruct((B,), jnp.float32),
        scratch_types=[
            pltpu.VMEM((_N_FIELDS, bpw), jnp.int32),
            pltpu.VMEM((_N_FIELDS, bpw), jnp.float32),
            pltpu.VMEM((bpw,), jnp.float32),
            pltpu.SemaphoreType.DMA,
        ],
    )
    def fo(sf_hbm, w_hbm, out_hbm, idx_v, vals_v, acc_v, sem):
        wid = lax.axis_index("s") * info.num_cores + lax.axis_index("c")
        base = wid * bpw
        pltpu.sync_copy(sf_hbm.at[:, pl.ds(base, bpw)], idx_v)
        cops = [
            pltpu.async_copy(w_hbm.at[idx_v.at[f]], vals_v.at[f], sem)
            for f in range(_N_FIELDS)
        ]
        for c in cops:
            c.wait()
        for c in range(bpw // _LANES):
            s = pl.ds(c * _LANES, _LANES)
            acc = vals_v[0, s]
            for f in range(1, _N_FIELDS):
                acc = acc + vals_v[f, s]
            acc_v[s] = acc
        pltpu.sync_copy(acc_v, out_hbm.at[pl.ds(base, bpw)])

    return fo(sf_t, w_flat)


_PAIRS = [
    (i, j) for i in range(_N_FIELDS) for j in range(i + 1, _N_FIELDS)
]


def _second_order_body2d(in_ref, out_ref):
    d = _EMBED_DIM
    fields = [in_ref[:, i * d : (i + 1) * d] for i in range(_N_FIELDS)]
    for p, (i, j) in enumerate(_PAIRS):
        out_ref[:, p * d : (p + 1) * d] = fields[i] * fields[j]


def _second_order_tc2d(embed_stack, tb):
    B = embed_stack.shape[0]
    in2 = embed_stack.reshape(B, _N_FIELDS * _EMBED_DIM)
    out2 = pl.pallas_call(
        _second_order_body2d,
        grid=(B // tb,),
        in_specs=[pl.BlockSpec((tb, _N_FIELDS * _EMBED_DIM), lambda b: (b, 0))],
        out_specs=pl.BlockSpec((tb, _N_PAIRS * _EMBED_DIM), lambda b: (b, 0)),
        out_shape=jax.ShapeDtypeStruct((B, _N_PAIRS * _EMBED_DIM), jnp.float32),
        compiler_params=pltpu.CompilerParams(
            dimension_semantics=("parallel",)
        ),
    )(in2)
    return out2.reshape(B, _N_PAIRS, _EMBED_DIM)


def _second_order_sc(embed2d, base_row, n_rows):
    d2 = _N_FIELDS * _EMBED_DIM
    dp = _N_PAIRS * _EMBED_DIM
    info = plsc.get_sparse_core_info()
    nw = info.num_cores * info.num_subcores
    bpw = n_rows // nw
    mesh = plsc.VectorSubcoreMesh(core_axis_name="c", subcore_axis_name="s")

    @functools.partial(
        pl.kernel,
        mesh=mesh,
        out_type=jax.ShapeDtypeStruct((n_rows, dp), jnp.float32),
        scratch_types=[
            pltpu.VMEM((1, d2), jnp.float32),
            pltpu.VMEM((1, dp), jnp.float32),
        ],
    )
    def so(emb_hbm, out_hbm, in_v, out_v):
        wid = lax.axis_index("s") * info.num_cores + lax.axis_index("c")
        row0 = wid * bpw

        def body(k, carry):
            pltpu.sync_copy(
                emb_hbm.at[pl.ds(base_row + row0 + k, 1), :], in_v
            )
            for c in range(_EMBED_DIM // _LANES):
                off = c * _LANES
                fields = [
                    in_v[0, pl.ds(f * _EMBED_DIM + off, _LANES)]
                    for f in range(_N_FIELDS)
                ]
                for p, (i, j) in enumerate(_PAIRS):
                    out_v[0, pl.ds(p * _EMBED_DIM + off, _LANES)] = (
                        fields[i] * fields[j]
                    )
            pltpu.sync_copy(out_v, out_hbm.at[pl.ds(row0 + k, 1), :])
            return carry

        lax.fori_loop(0, bpw, body, None)

    return so(embed2d)


def _second_order_tc_partial(in2, n_rows, tb):
    out2 = pl.pallas_call(
        _second_order_body2d,
        grid=(n_rows // tb,),
        in_specs=[pl.BlockSpec((tb, _N_FIELDS * _EMBED_DIM), lambda b: (b, 0))],
        out_specs=pl.BlockSpec((tb, _N_PAIRS * _EMBED_DIM), lambda b: (b, 0)),
        out_shape=jax.ShapeDtypeStruct((n_rows, _N_PAIRS * _EMBED_DIM), jnp.float32),
    )(in2)
    return out2


@jax.jit
def kernel(embed_stack, sparse_feat, w):
    B = embed_stack.shape[0]
    sf_t = sparse_feat.T
    w_flat = w.reshape(-1)
    first = _first_order_sc(sf_t, w_flat).reshape(B, 1)
    second = _second_order_tc2d(embed_stack, 256)
    return (first, second)


# speedup vs baseline: 2.3115x; 1.0004x over previous
import functools

import jax
import jax.numpy as jnp
from jax import lax
from jax.experimental import pallas as pl
from jax.experimental.pallas import tpu as pltpu
from jax.experimental.pallas import tpu_sc as plsc

_N_FIELDS = 26
_EMBED_DIM = 64
_N_PAIRS = (_N_FIELDS * (_N_FIELDS - 1)) // 2
_LANES = 16


def _first_order_sc(sf_t, w_flat):
    B = sf_t.shape[1]
    info = plsc.get_sparse_core_info()
    nw = info.num_cores * info.num_subcores
    bpw = B // nw
    mesh = plsc.VectorSubcoreMesh(core_axis_name="c", subcore_axis_name="s")

    @functools.partial(
        pl.kernel,
        mesh=mesh,
        out_type=jax.ShapeDtypeStruct((B,), jnp.float32),
        scratch_types=[
            pltpu.VMEM((_N_FIELDS, bpw), jnp.int32),
            pltpu.VMEM((_N_FIELDS, bpw), jnp.float32),
            pltpu.VMEM((bpw,), jnp.float32),
            pltpu.SemaphoreType.DMA,
        ],
    )
    def fo(sf_hbm, w_hbm, out_hbm, idx_v, vals_v, acc_v, sem):
        wid = lax.axis_index("s") * info.num_cores + lax.axis_index("c")
        base = wid * bpw
        pltpu.sync_copy(sf_hbm.at[:, pl.ds(base, bpw)], idx_v)
        cops = [
            pltpu.async_copy(w_hbm.at[idx_v.at[f]], vals_v.at[f], sem)
            for f in range(_N_FIELDS)
        ]
        for c in cops:
            c.wait()
        for c in range(bpw // _LANES):
            s = pl.ds(c * _LANES, _LANES)
            acc = vals_v[0, s]
            for f in range(1, _N_FIELDS):
                acc = acc + vals_v[f, s]
            acc_v[s] = acc
        pltpu.sync_copy(acc_v, out_hbm.at[pl.ds(base, bpw)])

    return fo(sf_t, w_flat)


_PAIRS = [
    (i, j) for i in range(_N_FIELDS) for j in range(i + 1, _N_FIELDS)
]


def _second_order_body2d(in_ref, out_ref):
    d = _EMBED_DIM
    fields = [in_ref[:, i * d : (i + 1) * d] for i in range(_N_FIELDS)]
    for p, (i, j) in enumerate(_PAIRS):
        out_ref[:, p * d : (p + 1) * d] = fields[i] * fields[j]


def _second_order_tc2d(embed_stack, tb):
    B = embed_stack.shape[0]
    in2 = embed_stack.reshape(B, _N_FIELDS * _EMBED_DIM)
    out2 = pl.pallas_call(
        _second_order_body2d,
        grid=(B // tb,),
        in_specs=[pl.BlockSpec((tb, _N_FIELDS * _EMBED_DIM), lambda b: (b, 0))],
        out_specs=pl.BlockSpec((tb, _N_PAIRS * _EMBED_DIM), lambda b: (b, 0)),
        out_shape=jax.ShapeDtypeStruct((B, _N_PAIRS * _EMBED_DIM), jnp.float32),
        compiler_params=pltpu.CompilerParams(
            dimension_semantics=("parallel",)
        ),
    )(in2)
    return out2.reshape(B, _N_PAIRS, _EMBED_DIM)


@jax.jit
def kernel(embed_stack, sparse_feat, w):
    B = embed_stack.shape[0]
    sf_t = sparse_feat.T
    w_flat = w.reshape(-1)
    first = _first_order_sc(sf_t, w_flat).reshape(B, 1)
    second = _second_order_tc2d(embed_stack, 128)
    return (first, second)
